# baseline (speedup 1.0000x reference)
import functools

import jax
import jax.numpy as jnp
from jax import lax
from jax.experimental import pallas as pl
from jax.experimental.pallas import tpu as pltpu
from jax.experimental.pallas import tpu_sc as plsc

_B = 16384
_D = 64
_NC = 2
_NS = 16
_NW = _NC * _NS
_BPW = _B // _NW
_CHUNK = 256
_NPASS = _BPW // _CHUNK
_NPRED = 201
_ROWS = 1000001
_TBLK = 2048


def _sc_body(const_hbm, pred_hbm, hidx_hbm, tidx_hbm, pidx_hbm, out_hbm,
             hidx_v, tidx_v, pidx_v, pred_l, head_v, tail_v, out_v, sem):
    wid = lax.axis_index("s") * _NC + lax.axis_index("c")
    base = wid * _BPW

    pltpu.sync_copy(hidx_hbm.at[pl.ds(base, _BPW)], hidx_v)
    pltpu.sync_copy(tidx_hbm.at[pl.ds(base, _BPW)], tidx_v)
    pltpu.sync_copy(pidx_hbm.at[pl.ds(base, _BPW)], pidx_v)
    pltpu.sync_copy(pred_hbm, pred_l)

    for p in range(_NPASS):
        off = p * _CHUNK

        def issue(i, carry):
            r0 = i * 16
            hv = hidx_v[pl.ds(off + r0, 16)]
            tv = tidx_v[pl.ds(off + r0, 16)]
            for j in range(16):
                pltpu.async_copy(const_hbm.at[hv[j]], head_v.at[r0 + j], sem)
                pltpu.async_copy(const_hbm.at[tv[j]], tail_v.at[r0 + j], sem)
            return carry

        lax.fori_loop(0, _CHUNK // 16, issue, 0)
        pltpu.make_async_copy(const_hbm.at[pl.ds(0, _CHUNK)], head_v, sem).wait()
        pltpu.make_async_copy(const_hbm.at[pl.ds(0, _CHUNK)], tail_v, sem).wait()

        def combine(i, carry):
            rv = pidx_v[pl.ds(off + i * 16, 16)]
            for j in range(16):
                r = i * 16 + j
                pj = rv[j]
                for k in range(_D // 16):
                    cs = pl.ds(16 * k, 16)
                    out_v[r, cs] = (head_v[r, cs] + pred_l[pj, cs]
                                    - tail_v[r, cs])
            return carry

        lax.fori_loop(0, _CHUNK // 16, combine, 0)

        pltpu.sync_copy(out_v, out_hbm.at[pl.ds(base + off, _CHUNK)])


def _tc_transpose_body(in_ref, out_ref):
    out_ref[...] = in_ref[...].T


def _tc_transpose(table_t):
    n_blk = (_ROWS + _TBLK - 1) // _TBLK
    return pl.pallas_call(
        _tc_transpose_body,
        grid=(n_blk,),
        in_specs=[pl.BlockSpec((_D, _TBLK), lambda i: (0, i))],
        out_specs=pl.BlockSpec((_TBLK, _D), lambda i: (i, 0)),
        out_shape=jax.ShapeDtypeStruct((_ROWS, _D), jnp.float32),
    )(table_t)


@jax.jit
def _run(hidx, tidx, pidx, const_table, pred_table):
    tbl = _tc_transpose(const_table.T)
    mesh = plsc.VectorSubcoreMesh(core_axis_name="c", subcore_axis_name="s")
    kfn = pl.kernel(
        _sc_body,
        out_type=jax.ShapeDtypeStruct((_B, _D), jnp.float32),
        mesh=mesh,
        scratch_types=[
            pltpu.VMEM((_BPW,), jnp.int32),
            pltpu.VMEM((_BPW,), jnp.int32),
            pltpu.VMEM((_BPW,), jnp.int32),
            pltpu.VMEM((_NPRED, _D), jnp.float32),
            pltpu.VMEM((_CHUNK, _D), jnp.float32),
            pltpu.VMEM((_CHUNK, _D), jnp.float32),
            pltpu.VMEM((_CHUNK, _D), jnp.float32),
            pltpu.SemaphoreType.DMA,
        ],
        compiler_params=pltpu.CompilerParams(needs_layout_passes=False),
    )
    return kfn(tbl, pred_table, hidx, tidx, pidx)


def kernel(predicate_indices, constant_indices, const_table, pred_table):
    hidx = constant_indices[:, 0]
    tidx = constant_indices[:, 1]
    pidx = predicate_indices[:, 0]
    return _run(hidx, tidx, pidx, const_table, pred_table)


# speedup vs baseline: 1.3196x; 1.3196x over previous
import functools

import jax
import jax.numpy as jnp
from jax import lax
from jax.experimental import pallas as pl
from jax.experimental.pallas import tpu as pltpu
from jax.experimental.pallas import tpu_sc as plsc

_B = 16384
_D = 64
_NC = 2
_NS = 16
_NW = _NC * _NS
_BPW = _B // _NW
_CHUNK = 256
_NPASS = _BPW // _CHUNK
_NPRED = 201


def _sc_body(const_hbm, pred_hbm, hidx_hbm, tidx_hbm, pidx_hbm, out_hbm,
             hidx_v, tidx_v, pidx_v, pred_l, head_v, tail_v, out_v, sem):
    wid = lax.axis_index("s") * _NC + lax.axis_index("c")
    base = wid * _BPW

    pltpu.sync_copy(hidx_hbm.at[pl.ds(base, _BPW)], hidx_v)
    pltpu.sync_copy(tidx_hbm.at[pl.ds(base, _BPW)], tidx_v)
    pltpu.sync_copy(pidx_hbm.at[pl.ds(base, _BPW)], pidx_v)
    pltpu.sync_copy(pred_hbm, pred_l)

    for p in range(_NPASS):
        off = p * _CHUNK

        def issue(i, carry):
            r0 = i * 16
            hv = hidx_v[pl.ds(off + r0, 16)]
            tv = tidx_v[pl.ds(off + r0, 16)]
            for j in range(16):
                pltpu.async_copy(const_hbm.at[hv[j]], head_v.at[r0 + j], sem)
                pltpu.async_copy(const_hbm.at[tv[j]], tail_v.at[r0 + j], sem)
            return carry

        lax.fori_loop(0, _CHUNK // 16, issue, 0)
        pltpu.make_async_copy(const_hbm.at[pl.ds(0, _CHUNK)], head_v, sem).wait()
        pltpu.make_async_copy(const_hbm.at[pl.ds(0, _CHUNK)], tail_v, sem).wait()

        def combine(i, carry):
            rv = pidx_v[pl.ds(off + i * 16, 16)]
            for j in range(16):
                r = i * 16 + j
                pj = rv[j]
                for k in range(_D // 16):
                    cs = pl.ds(16 * k, 16)
                    out_v[r, cs] = (head_v[r, cs] + pred_l[pj, cs]
                                    - tail_v[r, cs])
            return carry

        lax.fori_loop(0, _CHUNK // 16, combine, 0)

        pltpu.sync_copy(out_v, out_hbm.at[pl.ds(base + off, _CHUNK)])


@jax.jit
def _run(hidx, tidx, pidx, const_table, pred_table):
    mesh = plsc.VectorSubcoreMesh(core_axis_name="c", subcore_axis_name="s")
    kfn = pl.kernel(
        _sc_body,
        out_type=jax.ShapeDtypeStruct((_B, _D), jnp.float32),
        mesh=mesh,
        scratch_types=[
            pltpu.VMEM((_BPW,), jnp.int32),
            pltpu.VMEM((_BPW,), jnp.int32),
            pltpu.VMEM((_BPW,), jnp.int32),
            pltpu.VMEM((_NPRED, _D), jnp.float32),
            pltpu.VMEM((_CHUNK, _D), jnp.float32),
            pltpu.VMEM((_CHUNK, _D), jnp.float32),
            pltpu.VMEM((_CHUNK, _D), jnp.float32),
            pltpu.SemaphoreType.DMA,
        ],
        compiler_params=pltpu.CompilerParams(needs_layout_passes=False),
    )
    return kfn(const_table, pred_table, hidx, tidx, pidx)


def kernel(predicate_indices, constant_indices, const_table, pred_table):
    hidx = constant_indices[:, 0]
    tidx = constant_indices[:, 1]
    pidx = predicate_indices[:, 0]
    return _run(hidx, tidx, pidx, const_table, pred_table)
